# baseline (speedup 1.0000x reference)
import functools

import jax
import jax.numpy as jnp
from jax import lax
from jax.experimental import pallas as pl
from jax.experimental.pallas import tpu as pltpu
from jax.experimental.pallas import tpu_sc as plsc

_MU = 0.02
_BINS = 10
_LOSS_WEIGHT = 1.0

_N = 2_000_000
_ROWS = 15_625
_RW = 128
_R = 248
_NCHUNKS = (_ROWS - 1) // _R
_NW = 32


def _flatview(x):
    return jnp.flip(x.reshape(_ROWS, _RW), axis=1)


def _sc_histogram_pass(pred_flat, target_flat, weight_flat):
    mesh = plsc.VectorSubcoreMesh(core_axis_name="c", subcore_axis_name="s")

    @functools.partial(
        pl.kernel,
        mesh=mesh,
        out_type=(
            jax.ShapeDtypeStruct((_NW, _BINS * 16), jnp.float32),
            jax.ShapeDtypeStruct((_NW, _BINS * 16), jnp.float32),
            jax.ShapeDtypeStruct((_NW, 16), jnp.float32),
        ),
        scratch_types=[
            pltpu.VMEM((_R, _RW), jnp.float32),
            pltpu.VMEM((_R, _RW), jnp.float32),
            pltpu.VMEM((_R, _RW), jnp.float32),
            pltpu.VMEM((_BINS * 16,), jnp.float32),
            pltpu.VMEM((_BINS * 16,), jnp.float32),
            pltpu.VMEM((16,), jnp.float32),
        ],
        compiler_params=pltpu.CompilerParams(needs_layout_passes=False),
    )
    def k(pred_hbm, target_hbm, weight_hbm, cnt_hbm, sum_hbm, tw_hbm,
          pbuf, tbuf, wbuf, cnt_h, sum_h, tw_buf):
        wid = lax.axis_index("s") * 2 + lax.axis_index("c")
        zero16 = jnp.zeros((16,), jnp.float32)
        for b in range(_BINS):
            cnt_h[pl.ds(b * 16, 16)] = zero16
            sum_h[pl.ds(b * 16, 16)] = zero16

        lane = lax.iota(jnp.int32, 16)
        mu = jnp.float32(_MU)
        mu2 = jnp.float32(_MU * _MU)
        nchunks = (jnp.int32(_NCHUNKS) - wid + (_NW - 1)) // _NW

        def chunk_body(ci, tacc):
            roff = pl.multiple_of((wid + ci * _NW) * _R, 8)
            pltpu.sync_copy(pred_hbm.at[pl.ds(roff, _R), :], pbuf)
            pltpu.sync_copy(target_hbm.at[pl.ds(roff, _R), :], tbuf)
            pltpu.sync_copy(weight_hbm.at[pl.ds(roff, _R), :], wbuf)

            def row_body(r, acc):
                for c in range(_RW // 16):
                    sl = pl.ds(c * 16, 16)
                    p = pbuf[r, sl]
                    t = tbuf[r, sl]
                    w = wbuf[r, sl]
                    d = p - t
                    s = d * d + mu2
                    ibits = lax.bitcast_convert_type(s, jnp.int32)
                    seed = (jnp.int32(0x5F3759DF)
                            - lax.shift_right_logical(ibits, 1))
                    y = lax.bitcast_convert_type(seed, jnp.float32)
                    sh = jnp.float32(0.5) * s
                    y = y * (jnp.float32(1.5) - sh * y * y)
                    y = y * (jnp.float32(1.5) - sh * y * y)
                    loss = s * y - mu
                    g = jnp.abs(d) * y
                    validf = jnp.where(w > 0, jnp.float32(1.0),
                                       jnp.float32(0.0))
                    lwv = jnp.where(w > 0, loss * w, jnp.float32(0.0))
                    b = jnp.clip((g * jnp.float32(10.0)).astype(jnp.int32),
                                 0, 9)
                    idx = b * 16 + lane
                    plsc.addupdate_scatter(cnt_h, [idx], validf)
                    plsc.addupdate_scatter(sum_h, [idx], lwv)
                    acc = acc + w
                return acc

            return lax.fori_loop(0, _R, row_body, tacc)

        tacc = lax.fori_loop(0, nchunks, chunk_body, zero16)

        tw_buf[...] = tacc
        pltpu.sync_copy(cnt_h, cnt_hbm.at[wid])
        pltpu.sync_copy(sum_h, sum_hbm.at[wid])
        pltpu.sync_copy(tw_buf, tw_hbm.at[wid])

    return k(pred_flat, target_flat, weight_flat)


def _epilogue_body(cnt_ref, sum_ref, tw_ref, p_ref, t_ref, w_ref, o_ref):
    mu = jnp.float32(_MU)
    mu2 = jnp.float32(_MU * _MU)
    p = p_ref[...]
    t = t_ref[...]
    w = w_ref[...]
    d = p - t
    s = d * d + mu2
    sq = jnp.sqrt(s)
    loss = sq - mu
    g = jnp.abs(d) / sq
    validf = jnp.where(w > 0, 1.0, 0.0)
    lwv = loss * w * validf
    bidx = jnp.minimum((g * jnp.float32(10.0)).astype(jnp.int32), 9)

    tot = jnp.maximum(jnp.sum(tw_ref[...]) + jnp.sum(w), 1.0)
    r = jnp.float32(0.0)
    nbins = jnp.float32(0.0)
    for b in range(_BINS):
        inb = jnp.where(bidx == b, 1.0, 0.0)
        cb = jnp.sum(cnt_ref[:, b * 16:(b + 1) * 16]) + jnp.sum(inb * validf)
        sb = jnp.sum(sum_ref[:, b * 16:(b + 1) * 16]) + jnp.sum(inb * lwv)
        pos = cb > 0
        nbins = nbins + jnp.where(pos, 1.0, 0.0)
        r = r + jnp.where(pos, (tot / jnp.maximum(cb, 1.0)) * sb, 0.0)
    r = r / jnp.maximum(nbins, 1.0)
    o_ref[0, 0] = r * jnp.float32(_LOSS_WEIGHT / _N)


def kernel(pred, target, weight):
    pred_flat = _flatview(pred)
    target_flat = _flatview(target)
    weight_flat = _flatview(weight)
    cnt, s, tw = _sc_histogram_pass(pred_flat, target_flat, weight_flat)
    p_tail = pred_flat[_ROWS - 1:, :]
    t_tail = target_flat[_ROWS - 1:, :]
    w_tail = weight_flat[_ROWS - 1:, :]
    out = pl.pallas_call(
        _epilogue_body,
        out_shape=jax.ShapeDtypeStruct((1, 1), jnp.float32),
        out_specs=pl.BlockSpec(memory_space=pltpu.SMEM),
    )(cnt, s, tw, p_tail, t_tail, w_tail)
    return out[0, 0]


# speedup vs baseline: 1.0285x; 1.0285x over previous
import functools

import numpy as np

import jax
import jax.numpy as jnp
from jax import lax
from jax.experimental import pallas as pl
from jax.experimental.pallas import tpu as pltpu
from jax.experimental.pallas import tpu_sc as plsc

_MU = 0.02
_BINS = 10
_LOSS_WEIGHT = 1.0

_N = 2_000_000
_ROWS = 15_625
_RW = 128
_R = 248
_NCHUNKS = (_ROWS - 1) // _R
_NW = 32


def _flatview(x):
    perm = jnp.asarray(np.eye(_RW, dtype=np.float32)[:, ::-1])
    return jnp.dot(x.reshape(_ROWS, _RW), perm,
                   precision=lax.Precision.HIGHEST,
                   preferred_element_type=jnp.float32)


def _sc_histogram_pass(pred_flat, target_flat, weight_flat):
    mesh = plsc.VectorSubcoreMesh(core_axis_name="c", subcore_axis_name="s")

    @functools.partial(
        pl.kernel,
        mesh=mesh,
        out_type=(
            jax.ShapeDtypeStruct((_NW, _BINS * 16), jnp.float32),
            jax.ShapeDtypeStruct((_NW, _BINS * 16), jnp.float32),
            jax.ShapeDtypeStruct((_NW, 16), jnp.float32),
        ),
        scratch_types=[
            pltpu.VMEM((_R, _RW), jnp.float32),
            pltpu.VMEM((_R, _RW), jnp.float32),
            pltpu.VMEM((_R, _RW), jnp.float32),
            pltpu.VMEM((_BINS * 16,), jnp.float32),
            pltpu.VMEM((_BINS * 16,), jnp.float32),
            pltpu.VMEM((16,), jnp.float32),
        ],
        compiler_params=pltpu.CompilerParams(needs_layout_passes=False),
    )
    def k(pred_hbm, target_hbm, weight_hbm, cnt_hbm, sum_hbm, tw_hbm,
          pbuf, tbuf, wbuf, cnt_h, sum_h, tw_buf):
        wid = lax.axis_index("s") * 2 + lax.axis_index("c")
        zero16 = jnp.zeros((16,), jnp.float32)
        for b in range(_BINS):
            cnt_h[pl.ds(b * 16, 16)] = zero16
            sum_h[pl.ds(b * 16, 16)] = zero16

        lane = lax.iota(jnp.int32, 16)
        mu = jnp.float32(_MU)
        mu2 = jnp.float32(_MU * _MU)
        nchunks = (jnp.int32(_NCHUNKS) - wid + (_NW - 1)) // _NW

        def chunk_body(ci, tacc):
            roff = pl.multiple_of((wid + ci * _NW) * _R, 8)
            pltpu.sync_copy(pred_hbm.at[pl.ds(roff, _R), :], pbuf)
            pltpu.sync_copy(target_hbm.at[pl.ds(roff, _R), :], tbuf)
            pltpu.sync_copy(weight_hbm.at[pl.ds(roff, _R), :], wbuf)

            def row_body(r, acc):
                for c in range(_RW // 16):
                    sl = pl.ds(c * 16, 16)
                    p = pbuf[r, sl]
                    t = tbuf[r, sl]
                    w = wbuf[r, sl]
                    d = p - t
                    s = d * d + mu2
                    ibits = lax.bitcast_convert_type(s, jnp.int32)
                    seed = (jnp.int32(0x5F3759DF)
                            - lax.shift_right_logical(ibits, 1))
                    y = lax.bitcast_convert_type(seed, jnp.float32)
                    sh = jnp.float32(0.5) * s
                    y = y * (jnp.float32(1.5) - sh * y * y)
                    y = y * (jnp.float32(1.5) - sh * y * y)
                    loss = s * y - mu
                    g = jnp.abs(d) * y
                    validf = jnp.where(w > 0, jnp.float32(1.0),
                                       jnp.float32(0.0))
                    lwv = jnp.where(w > 0, loss * w, jnp.float32(0.0))
                    b = jnp.clip((g * jnp.float32(10.0)).astype(jnp.int32),
                                 0, 9)
                    idx = b * 16 + lane
                    plsc.addupdate_scatter(cnt_h, [idx], validf)
                    plsc.addupdate_scatter(sum_h, [idx], lwv)
                    acc = acc + w
                return acc

            return lax.fori_loop(0, _R, row_body, tacc)

        tacc = lax.fori_loop(0, nchunks, chunk_body, zero16)

        tw_buf[...] = tacc
        pltpu.sync_copy(cnt_h, cnt_hbm.at[wid])
        pltpu.sync_copy(sum_h, sum_hbm.at[wid])
        pltpu.sync_copy(tw_buf, tw_hbm.at[wid])

    return k(pred_flat, target_flat, weight_flat)


def _epilogue_body(cnt_ref, sum_ref, tw_ref, p_ref, t_ref, w_ref, o_ref):
    mu = jnp.float32(_MU)
    mu2 = jnp.float32(_MU * _MU)
    p = p_ref[...]
    t = t_ref[...]
    w = w_ref[...]
    d = p - t
    s = d * d + mu2
    sq = jnp.sqrt(s)
    loss = sq - mu
    g = jnp.abs(d) / sq
    validf = jnp.where(w > 0, 1.0, 0.0)
    lwv = loss * w * validf
    bidx = jnp.minimum((g * jnp.float32(10.0)).astype(jnp.int32), 9)

    tot = jnp.maximum(jnp.sum(tw_ref[...]) + jnp.sum(w), 1.0)
    r = jnp.float32(0.0)
    nbins = jnp.float32(0.0)
    for b in range(_BINS):
        inb = jnp.where(bidx == b, 1.0, 0.0)
        cb = jnp.sum(cnt_ref[:, b * 16:(b + 1) * 16]) + jnp.sum(inb * validf)
        sb = jnp.sum(sum_ref[:, b * 16:(b + 1) * 16]) + jnp.sum(inb * lwv)
        pos = cb > 0
        nbins = nbins + jnp.where(pos, 1.0, 0.0)
        r = r + jnp.where(pos, (tot / jnp.maximum(cb, 1.0)) * sb, 0.0)
    r = r / jnp.maximum(nbins, 1.0)
    o_ref[0, 0] = r * jnp.float32(_LOSS_WEIGHT / _N)


def kernel(pred, target, weight):
    pred_flat = _flatview(pred)
    target_flat = _flatview(target)
    weight_flat = _flatview(weight)
    cnt, s, tw = _sc_histogram_pass(pred_flat, target_flat, weight_flat)
    p_tail = pred_flat[_ROWS - 1:, :]
    t_tail = target_flat[_ROWS - 1:, :]
    w_tail = weight_flat[_ROWS - 1:, :]
    out = pl.pallas_call(
        _epilogue_body,
        out_shape=jax.ShapeDtypeStruct((1, 1), jnp.float32),
        out_specs=pl.BlockSpec(memory_space=pltpu.SMEM),
    )(cnt, s, tw, p_tail, t_tail, w_tail)
    return out[0, 0]


# speedup vs baseline: 6.4303x; 6.2524x over previous
import functools

import jax
import jax.numpy as jnp
from jax import lax
from jax.experimental import pallas as pl
from jax.experimental.pallas import tpu as pltpu
from jax.experimental.pallas import tpu_sc as plsc

_MU = 0.02
_BINS = 10
_LOSS_WEIGHT = 1.0

_N = 2_000_000
_NR = 500_000
_R = 400
_NCH = _NR // _R
_VPC = _R * 4 // 16
_NW = 32
_PAIRS = 20


def _sc_histogram_pass(ptw):
    mesh = plsc.VectorSubcoreMesh(core_axis_name="c", subcore_axis_name="s")

    @functools.partial(
        pl.kernel,
        mesh=mesh,
        out_type=(
            jax.ShapeDtypeStruct((_NW, _BINS * 16), jnp.float32),
            jax.ShapeDtypeStruct((_NW, _BINS * 16), jnp.float32),
            jax.ShapeDtypeStruct((_NW, 16), jnp.float32),
        ),
        scratch_types=[
            pltpu.VMEM((2 * _R, 12), jnp.float32),
            pltpu.VMEM((_BINS * 16,), jnp.float32),
            pltpu.VMEM((_BINS * 16,), jnp.float32),
            pltpu.VMEM((16,), jnp.float32),
            pltpu.SemaphoreType.DMA,
            pltpu.SemaphoreType.DMA,
        ],
        compiler_params=pltpu.CompilerParams(needs_layout_passes=False),
    )
    def k(ptw_hbm, cnt_hbm, sum_hbm, tw_hbm,
          buf, cnt_h, sum_h, tw_buf, sem0, sem1):
        wid = lax.axis_index("s") * 2 + lax.axis_index("c")
        zero16 = jnp.zeros((16,), jnp.float32)
        for b in range(_BINS):
            cnt_h[pl.ds(b * 16, 16)] = zero16
            sum_h[pl.ds(b * 16, 16)] = zero16
        tw_buf[...] = zero16

        lane = lax.iota(jnp.int32, 16)
        rowpat = lax.shift_right_logical(lane, 2)
        colpat = lax.bitwise_and(lane, 3)
        mu = jnp.float32(_MU)
        mu2 = jnp.float32(_MU * _MU)

        def copy(gc, slot, sem):
            roff = pl.multiple_of(gc * _R, 8)
            dst = pl.ds(slot * _R, _R)
            return pltpu.make_async_copy(
                ptw_hbm.at[pl.ds(roff, _R), :], buf.at[dst, :], sem)

        def issue(gc, slot, sem):
            copy(gc, slot, sem).start()

        def drain(gc, slot, sem):
            copy(gc, slot, sem).wait()

        def compute(slot):
            base = slot * _R

            def vreg_body(i, carry):
                ridx = base + i * 4 + rowpat
                p = plsc.load_gather(buf, [ridx, colpat])
                t = plsc.load_gather(buf, [ridx, colpat + 4])
                w = plsc.load_gather(buf, [ridx, colpat + 8])
                d = p - t
                s = d * d + mu2
                ibits = lax.bitcast_convert_type(s, jnp.int32)
                seed = (jnp.int32(0x5F3759DF)
                        - lax.shift_right_logical(ibits, 1))
                y = lax.bitcast_convert_type(seed, jnp.float32)
                sh = jnp.float32(0.5) * s
                y = y * (jnp.float32(1.5) - sh * y * y)
                y = y * (jnp.float32(1.5) - sh * y * y)
                loss = s * y - mu
                g = jnp.abs(d) * y
                validf = jnp.where(w > 0, jnp.float32(1.0), jnp.float32(0.0))
                lwv = jnp.where(w > 0, loss * w, jnp.float32(0.0))
                b = jnp.clip((g * jnp.float32(10.0)).astype(jnp.int32), 0, 9)
                idx = b * 16 + lane
                plsc.addupdate_scatter(cnt_h, [idx], validf)
                plsc.addupdate_scatter(sum_h, [idx], lwv)
                plsc.addupdate(tw_buf.at[...], w)
                return carry

            lax.fori_loop(0, _VPC, vreg_body, 0)

        issue(wid, 0, sem0)

        def pair_body(i, carry):
            gc0 = wid + i * 2 * _NW
            gc1 = gc0 + _NW

            @pl.when(gc1 < _NCH)
            def _():
                issue(gc1, 1, sem1)

            @pl.when(gc0 < _NCH)
            def _():
                drain(gc0, 0, sem0)
                compute(0)

            @pl.when(gc0 + 2 * _NW < _NCH)
            def _():
                issue(gc0 + 2 * _NW, 0, sem0)

            @pl.when(gc1 < _NCH)
            def _():
                drain(gc1, 1, sem1)
                compute(1)

            return carry

        lax.fori_loop(0, _PAIRS, pair_body, 0)

        pltpu.sync_copy(cnt_h, cnt_hbm.at[wid])
        pltpu.sync_copy(sum_h, sum_hbm.at[wid])
        pltpu.sync_copy(tw_buf, tw_hbm.at[wid])

    return k(ptw)


def _epilogue_body(cnt_ref, sum_ref, tw_ref, o_ref):
    tot = jnp.maximum(jnp.sum(tw_ref[...]), 1.0)
    r = jnp.float32(0.0)
    nbins = jnp.float32(0.0)
    for b in range(_BINS):
        cb = jnp.sum(cnt_ref[:, b * 16:(b + 1) * 16])
        sb = jnp.sum(sum_ref[:, b * 16:(b + 1) * 16])
        pos = cb > 0
        nbins = nbins + jnp.where(pos, 1.0, 0.0)
        r = r + jnp.where(pos, (tot / jnp.maximum(cb, 1.0)) * sb, 0.0)
    r = r / jnp.maximum(nbins, 1.0)
    o_ref[0, 0] = r * jnp.float32(_LOSS_WEIGHT / _N)


def kernel(pred, target, weight):
    ptw = jnp.concatenate([pred, target, weight], axis=1)
    cnt, s, tw = _sc_histogram_pass(ptw)
    out = pl.pallas_call(
        _epilogue_body,
        out_shape=jax.ShapeDtypeStruct((1, 1), jnp.float32),
        out_specs=pl.BlockSpec(memory_space=pltpu.SMEM),
    )(cnt, s, tw)
    return out[0, 0]
